# baseline (speedup 1.0000x reference)
import functools
import math

import jax
import jax.numpy as jnp
from jax import lax
from jax.experimental import pallas as pl
from jax.experimental.pallas import tpu as pltpu
from jax.experimental.pallas import tpu_sc as plsc

K = 30
V = 32
B = 65536
N = B * V
GP = 512
MIN_BIN_WIDTH = 1e-3
MIN_BIN_HEIGHT = 1e-3
MIN_DERIVATIVE = 1e-3
EPS = 1e-6
LN2 = 0.6931471805599453
LNC = (3.511021356650268e-06, 0.9997923620654495, -0.49697743071907685,
       0.31458917398920905, -0.1887808235491981, 0.08172564529133709,
       -0.01720779923058697)

NW = 32
W = 128
R = N // W
RT = R // NW
RS = 64
S = RS * W
N_STAGES = RT // RS


def _tables_body(uw_ref, uh_ref, ud_ref,
                 locs_ref, cw_ref, rw_ref, ch_ref, h_ref, d_ref, grid_ref):
    uw = uw_ref[...]
    uh = uh_ref[...]
    ud = ud_ref[...]

    rj = lax.broadcasted_iota(jnp.int32, (K, K + 1), 0)
    ck = lax.broadcasted_iota(jnp.int32, (K, K + 1), 1)
    tri = (rj < ck).astype(jnp.float32)

    col31 = lax.broadcasted_iota(jnp.int32, (V, K + 1), 1)

    def knots(u):
        m = jnp.max(u, axis=-1, keepdims=True)
        e = jnp.exp(u - m)
        sm = e / jnp.sum(e, axis=-1, keepdims=True)
        frac = MIN_BIN_WIDTH + (1.0 - MIN_BIN_WIDTH * K) * sm
        c = lax.dot_general(frac, tri, (((1,), (0,)), ((), ())),
                            precision=lax.Precision.HIGHEST,
                            preferred_element_type=jnp.float32)
        c = jnp.where(col31 == K, 1.0, c)
        return c, c[:, 1:] - c[:, :-1]

    cw, w = knots(uw)
    ch, h = knots(uh)

    const = math.log(math.exp(1.0 - MIN_DERIVATIVE) - 1.0)
    ud_p = jnp.concatenate(
        [jnp.full((V, 1), const, jnp.float32), ud,
         jnp.full((V, 1), const, jnp.float32)], axis=1)
    deriv = MIN_DERIVATIVE + (jnp.log1p(jnp.exp(-jnp.abs(ud_p)))
                              + jnp.maximum(ud_p, 0.0))

    locs = jnp.where(col31 == K, 1.0 + EPS, cw)

    gp = lax.broadcasted_iota(jnp.int32, (V, GP), 1).astype(jnp.float32)
    gp = gp * (1.0 / GP)
    g = jnp.zeros((V, GP), jnp.int32)
    for k in range(K + 1):
        g = g + (gp >= locs[:, k:k + 1]).astype(jnp.int32)
    g = g - 1
    rows = lax.broadcasted_iota(jnp.int32, (V, GP), 0)
    grid_ref[...] = g * V + rows

    locs_ref[...] = jnp.concatenate(
        [locs, jnp.full((V, 1), 2.0, jnp.float32)], 1)
    cw_ref[...] = jnp.concatenate([cw, jnp.ones((V, 1), jnp.float32)], 1)
    ch_ref[...] = jnp.concatenate([ch, jnp.ones((V, 1), jnp.float32)], 1)
    rw_ref[...] = jnp.concatenate([1.0 / w, jnp.ones((V, 2), jnp.float32)], 1)
    h_ref[...] = jnp.concatenate([h, jnp.ones((V, 2), jnp.float32)], 1)
    d_ref[...] = jnp.concatenate([deriv, jnp.ones((V, 1), jnp.float32)], 1)


_t32 = jax.ShapeDtypeStruct((V, V), jnp.float32)


def _make_tables(uw, uh, ud):
    return pl.pallas_call(
        _tables_body,
        out_shape=(_t32,) * 6 + (jax.ShapeDtypeStruct((V, GP), jnp.int32),),
    )(uw, uh, ud)


def _ln16(r):
    bits = plsc.bitcast(r, jnp.int32)
    e = lax.shift_right_arithmetic(bits, 23) - 127
    m = plsc.bitcast((bits & 0x007FFFFF) | 0x3F800000, jnp.float32)
    t = m - 1.0
    p = jnp.full((16,), LNC[6], jnp.float32)
    for c in LNC[5::-1]:
        p = p * t + c
    return e.astype(jnp.float32) * LN2 + p


def _sc_body(x_hbm, locs_hbm, cw_hbm, rw_hbm, ch_hbm, h_hbm, d_hbm, grid_hbm,
             out_hbm, ld_hbm,
             x_v0, x_v1, o_v0, o_v1, l_v0, l_v1,
             locs_v, cw_v, rw_v, ch_v, h_v, d_v, grid_v,
             sem_x0, sem_x1, sem_o0, sem_o1, sem_l0, sem_l1):
    wid = lax.axis_index("s") * 2 + lax.axis_index("c")
    base = wid * RT

    x_bufs = (x_v0, x_v1)
    o_bufs = (o_v0, o_v1)
    l_bufs = (l_v0, l_v1)
    sx = (sem_x0, sem_x1)
    so = (sem_o0, sem_o1)
    sl = (sem_l0, sem_l1)

    h_x, h_o, h_l = {}, {}, {}
    h_x[0] = pltpu.async_copy(x_hbm.at[pl.ds(base, RS)], x_bufs[0], sx[0])

    pltpu.sync_copy(locs_hbm, locs_v)
    pltpu.sync_copy(cw_hbm, cw_v)
    pltpu.sync_copy(rw_hbm, rw_v)
    pltpu.sync_copy(ch_hbm, ch_v)
    pltpu.sync_copy(h_hbm, h_v)
    pltpu.sync_copy(d_hbm, d_v)
    pltpu.sync_copy(grid_hbm, grid_v)

    iota16 = lax.iota(jnp.int32, 16)

    for st in range(N_STAGES):
        bi = st % 2
        off = base + st * RS
        if st + 1 < N_STAGES:
            nb = (st + 1) % 2
            h_x[st + 1] = pltpu.async_copy(
                x_hbm.at[pl.ds(off + RS, RS)], x_bufs[nb], sx[nb])
        h_x[st].wait()
        if st >= 2:
            h_o[st - 2].wait()
            h_l[st - 2].wait()
        x_v = x_bufs[bi]
        o_v = o_bufs[bi]
        l_v = l_bufs[bi]

        @plsc.parallel_loop(0, S, step=16, unroll=4)
        def inner(i):
            r = i >> 7
            c = i & 127
            x = x_v[r, pl.ds(c, 16)]
            v = iota16 + (i & 16)

            s = (x * float(GP)).astype(jnp.int32)
            lo0 = plsc.load_gather(grid_v, [(s << 5) + v])
            t2 = lo0 + 2 * V
            g2 = plsc.load_gather(locs_v, [t2])
            lo = jnp.where(x >= g2, t2, lo0)
            t1 = lo + V
            g1 = plsc.load_gather(locs_v, [t1])
            lo = jnp.where(x >= g1, t1, lo)

            icw = plsc.load_gather(cw_v, [lo])
            irw = plsc.load_gather(rw_v, [lo])
            ich = plsc.load_gather(ch_v, [lo])
            ih = plsc.load_gather(h_v, [lo])
            id0 = plsc.load_gather(d_v, [lo])
            id1 = plsc.load_gather(d_v, [lo + V])

            idl = ih * irw
            th = (x - icw) * irw
            th2 = th * th
            omt = 1.0 - th
            tomt = th * omt
            num = ih * (idl * th2 + id0 * tomt)
            den = idl + (id0 + id1 - 2.0 * idl) * tomt
            rden = 1.0 / den
            dn = (idl * idl) * (id1 * th2 + 2.0 * idl * tomt
                                + id0 * (omt * omt))
            o_v[r, pl.ds(c, 16)] = ich + num * rden
            l_v[r, pl.ds(c, 16)] = _ln16(dn * rden * rden)

        h_o[st] = pltpu.async_copy(o_v, out_hbm.at[pl.ds(off, RS)], so[bi])
        h_l[st] = pltpu.async_copy(l_v, ld_hbm.at[pl.ds(off, RS)], sl[bi])

    for st in range(max(0, N_STAGES - 2), N_STAGES):
        h_o[st].wait()
        h_l[st].wait()


_sc_call = functools.partial(
    pl.kernel,
    mesh=plsc.VectorSubcoreMesh(core_axis_name="c", subcore_axis_name="s"),
    compiler_params=pltpu.CompilerParams(needs_layout_passes=False),
    out_type=(jax.ShapeDtypeStruct((R, W), jnp.float32),
              jax.ShapeDtypeStruct((R, W), jnp.float32)),
    scratch_types=[
        pltpu.VMEM((RS, W), jnp.float32),
        pltpu.VMEM((RS, W), jnp.float32),
        pltpu.VMEM((RS, W), jnp.float32),
        pltpu.VMEM((RS, W), jnp.float32),
        pltpu.VMEM((RS, W), jnp.float32),
        pltpu.VMEM((RS, W), jnp.float32),
        pltpu.VMEM((V * V,), jnp.float32),
        pltpu.VMEM((V * V,), jnp.float32),
        pltpu.VMEM((V * V,), jnp.float32),
        pltpu.VMEM((V * V,), jnp.float32),
        pltpu.VMEM((V * V,), jnp.float32),
        pltpu.VMEM((V * V,), jnp.float32),
        pltpu.VMEM((V * GP,), jnp.int32),
        pltpu.SemaphoreType.DMA,
        pltpu.SemaphoreType.DMA,
        pltpu.SemaphoreType.DMA,
        pltpu.SemaphoreType.DMA,
        pltpu.SemaphoreType.DMA,
        pltpu.SemaphoreType.DMA,
    ],
)(_sc_body)


def kernel(inputs, unnormalized_widths, unnormalized_heights,
           unnormalized_derivatives):
    tables = _make_tables(
        unnormalized_widths, unnormalized_heights, unnormalized_derivatives)
    out, ld = _sc_call(inputs.reshape(R, W), *(t.T.reshape(-1) for t in tables))
    return out.reshape(B, V), ld.reshape(B, V)


# speedup vs baseline: 1.1328x; 1.1328x over previous
import functools
import math

import jax
import jax.numpy as jnp
from jax import lax
from jax.experimental import pallas as pl
from jax.experimental.pallas import tpu as pltpu
from jax.experimental.pallas import tpu_sc as plsc

K = 30
V = 32
B = 65536
N = B * V
GP = 512
MIN_BIN_WIDTH = 1e-3
MIN_BIN_HEIGHT = 1e-3
MIN_DERIVATIVE = 1e-3
EPS = 1e-6
LN2 = 0.6931471805599453
LNC = (3.511021356650268e-06, 0.9997923620654495, -0.49697743071907685,
       0.31458917398920905, -0.1887808235491981, 0.08172564529133709,
       -0.01720779923058697)

NW = 32
RT = B // NW
RS = 128
S = RS * V
N_STAGES = RT // RS


def _tables_body(uw_ref, uh_ref, ud_ref,
                 locs_ref, cw_ref, rw_ref, ch_ref, h_ref, d_ref, grid_ref):
    uw = uw_ref[...]
    uh = uh_ref[...]
    ud = ud_ref[...]

    rj = lax.broadcasted_iota(jnp.int32, (K, K + 1), 0)
    ck = lax.broadcasted_iota(jnp.int32, (K, K + 1), 1)
    tri = (rj < ck).astype(jnp.float32)

    col31 = lax.broadcasted_iota(jnp.int32, (V, K + 1), 1)

    def knots(u):
        m = jnp.max(u, axis=-1, keepdims=True)
        e = jnp.exp(u - m)
        sm = e / jnp.sum(e, axis=-1, keepdims=True)
        frac = MIN_BIN_WIDTH + (1.0 - MIN_BIN_WIDTH * K) * sm
        c = lax.dot_general(frac, tri, (((1,), (0,)), ((), ())),
                            precision=lax.Precision.HIGHEST,
                            preferred_element_type=jnp.float32)
        c = jnp.where(col31 == K, 1.0, c)
        return c, c[:, 1:] - c[:, :-1]

    cw, w = knots(uw)
    ch, h = knots(uh)

    const = math.log(math.exp(1.0 - MIN_DERIVATIVE) - 1.0)
    ud_p = jnp.concatenate(
        [jnp.full((V, 1), const, jnp.float32), ud,
         jnp.full((V, 1), const, jnp.float32)], axis=1)
    deriv = MIN_DERIVATIVE + (jnp.log1p(jnp.exp(-jnp.abs(ud_p)))
                              + jnp.maximum(ud_p, 0.0))

    locs = jnp.where(col31 == K, 1.0 + EPS, cw)

    gp = lax.broadcasted_iota(jnp.int32, (V, GP), 1).astype(jnp.float32)
    gp = gp * (1.0 / GP)
    g = jnp.zeros((V, GP), jnp.int32)
    for k in range(K + 1):
        g = g + (gp >= locs[:, k:k + 1]).astype(jnp.int32)
    g = g - 1
    rows = lax.broadcasted_iota(jnp.int32, (V, GP), 0)
    grid_ref[...] = g * V + rows

    locs_ref[...] = jnp.concatenate(
        [locs, jnp.full((V, 1), 2.0, jnp.float32)], 1)
    cw_ref[...] = jnp.concatenate([cw, jnp.ones((V, 1), jnp.float32)], 1)
    ch_ref[...] = jnp.concatenate([ch, jnp.ones((V, 1), jnp.float32)], 1)
    rw_ref[...] = jnp.concatenate([1.0 / w, jnp.ones((V, 2), jnp.float32)], 1)
    h_ref[...] = jnp.concatenate([h, jnp.ones((V, 2), jnp.float32)], 1)
    d_ref[...] = jnp.concatenate([deriv, jnp.ones((V, 1), jnp.float32)], 1)


_t32 = jax.ShapeDtypeStruct((V, V), jnp.float32)


def _make_tables(uw, uh, ud):
    return pl.pallas_call(
        _tables_body,
        out_shape=(_t32,) * 6 + (jax.ShapeDtypeStruct((V, GP), jnp.int32),),
    )(uw, uh, ud)


def _ln16(r):
    bits = plsc.bitcast(r, jnp.int32)
    e = lax.shift_right_arithmetic(bits, 23) - 127
    m = plsc.bitcast((bits & 0x007FFFFF) | 0x3F800000, jnp.float32)
    t = m - 1.0
    p = jnp.full((16,), LNC[6], jnp.float32)
    for c in LNC[5::-1]:
        p = p * t + c
    return e.astype(jnp.float32) * LN2 + p


def _sc_body(x_hbm, locs_hbm, cw_hbm, rw_hbm, ch_hbm, h_hbm, d_hbm, grid_hbm,
             out_hbm, ld_hbm,
             x_v0, x_v1, o_v0, o_v1, l_v0, l_v1,
             locs_v, cw_v, rw_v, ch_v, h_v, d_v, grid_v,
             sem_x0, sem_x1, sem_o0, sem_o1, sem_l0, sem_l1):
    wid = lax.axis_index("s") * 2 + lax.axis_index("c")
    base = wid * RT

    x_bufs = (x_v0, x_v1)
    o_bufs = (o_v0, o_v1)
    l_bufs = (l_v0, l_v1)
    sx = (sem_x0, sem_x1)
    so = (sem_o0, sem_o1)
    sl = (sem_l0, sem_l1)

    h_x, h_o, h_l = {}, {}, {}
    h_x[0] = pltpu.async_copy(x_hbm.at[pl.ds(base, RS)], x_bufs[0], sx[0])

    pltpu.sync_copy(locs_hbm, locs_v)
    pltpu.sync_copy(cw_hbm, cw_v)
    pltpu.sync_copy(rw_hbm, rw_v)
    pltpu.sync_copy(ch_hbm, ch_v)
    pltpu.sync_copy(h_hbm, h_v)
    pltpu.sync_copy(d_hbm, d_v)
    pltpu.sync_copy(grid_hbm, grid_v)

    iota16 = lax.iota(jnp.int32, 16)

    for st in range(N_STAGES):
        bi = st % 2
        off = base + st * RS
        if st + 1 < N_STAGES:
            nb = (st + 1) % 2
            h_x[st + 1] = pltpu.async_copy(
                x_hbm.at[pl.ds(off + RS, RS)], x_bufs[nb], sx[nb])
        h_x[st].wait()
        if st >= 2:
            h_o[st - 2].wait()
            h_l[st - 2].wait()
        x_v = x_bufs[bi]
        o_v = o_bufs[bi]
        l_v = l_bufs[bi]

        @plsc.parallel_loop(0, S, step=16, unroll=4)
        def inner(i):
            r = i >> 5
            c = i & 16
            x = x_v[r, pl.ds(c, 16)]
            v = iota16 + c

            s = (x * float(GP)).astype(jnp.int32)
            lo0 = plsc.load_gather(grid_v, [(s << 5) + v])
            t2 = lo0 + 2 * V
            g2 = plsc.load_gather(locs_v, [t2])
            lo = jnp.where(x >= g2, t2, lo0)
            t1 = lo + V
            g1 = plsc.load_gather(locs_v, [t1])
            lo = jnp.where(x >= g1, t1, lo)

            icw = plsc.load_gather(cw_v, [lo])
            irw = plsc.load_gather(rw_v, [lo])
            ich = plsc.load_gather(ch_v, [lo])
            ih = plsc.load_gather(h_v, [lo])
            id0 = plsc.load_gather(d_v, [lo])
            id1 = plsc.load_gather(d_v, [lo + V])

            idl = ih * irw
            th = (x - icw) * irw
            th2 = th * th
            omt = 1.0 - th
            tomt = th * omt
            num = ih * (idl * th2 + id0 * tomt)
            den = idl + (id0 + id1 - 2.0 * idl) * tomt
            rden = 1.0 / den
            dn = (idl * idl) * (id1 * th2 + 2.0 * idl * tomt
                                + id0 * (omt * omt))
            o_v[r, pl.ds(c, 16)] = ich + num * rden
            l_v[r, pl.ds(c, 16)] = _ln16(dn * rden * rden)

        h_o[st] = pltpu.async_copy(o_v, out_hbm.at[pl.ds(off, RS)], so[bi])
        h_l[st] = pltpu.async_copy(l_v, ld_hbm.at[pl.ds(off, RS)], sl[bi])

    for st in range(max(0, N_STAGES - 2), N_STAGES):
        h_o[st].wait()
        h_l[st].wait()


_sc_call = functools.partial(
    pl.kernel,
    mesh=plsc.VectorSubcoreMesh(core_axis_name="c", subcore_axis_name="s"),
    compiler_params=pltpu.CompilerParams(needs_layout_passes=False),
    out_type=(jax.ShapeDtypeStruct((B, V), jnp.float32),
              jax.ShapeDtypeStruct((B, V), jnp.float32)),
    scratch_types=[
        pltpu.VMEM((RS, V), jnp.float32),
        pltpu.VMEM((RS, V), jnp.float32),
        pltpu.VMEM((RS, V), jnp.float32),
        pltpu.VMEM((RS, V), jnp.float32),
        pltpu.VMEM((RS, V), jnp.float32),
        pltpu.VMEM((RS, V), jnp.float32),
        pltpu.VMEM((V * V,), jnp.float32),
        pltpu.VMEM((V * V,), jnp.float32),
        pltpu.VMEM((V * V,), jnp.float32),
        pltpu.VMEM((V * V,), jnp.float32),
        pltpu.VMEM((V * V,), jnp.float32),
        pltpu.VMEM((V * V,), jnp.float32),
        pltpu.VMEM((V * GP,), jnp.int32),
        pltpu.SemaphoreType.DMA,
        pltpu.SemaphoreType.DMA,
        pltpu.SemaphoreType.DMA,
        pltpu.SemaphoreType.DMA,
        pltpu.SemaphoreType.DMA,
        pltpu.SemaphoreType.DMA,
    ],
)(_sc_body)


def kernel(inputs, unnormalized_widths, unnormalized_heights,
           unnormalized_derivatives):
    tables = _make_tables(
        unnormalized_widths, unnormalized_heights, unnormalized_derivatives)
    return _sc_call(inputs, *(t.T.reshape(-1) for t in tables))


# speedup vs baseline: 1.1811x; 1.0427x over previous
import functools
import math

import jax
import jax.numpy as jnp
from jax import lax
from jax.experimental import pallas as pl
from jax.experimental.pallas import tpu as pltpu
from jax.experimental.pallas import tpu_sc as plsc

K = 30
V = 32
B = 65536
N = B * V
GP = 512
MIN_BIN_WIDTH = 1e-3
MIN_BIN_HEIGHT = 1e-3
MIN_DERIVATIVE = 1e-3
EPS = 1e-6
LN2 = 0.6931471805599453
LNC = (3.511021356650268e-06, 0.9997923620654495, -0.49697743071907685,
       0.31458917398920905, -0.1887808235491981, 0.08172564529133709,
       -0.01720779923058697)

NW = 32
RT = B // NW
RS = 128
S = RS * V
N_STAGES = RT // RS


def _tables_body(uw_ref, uh_ref, ud_ref,
                 locs_ref, cw_ref, rw_ref, ch_ref, h_ref, d_ref, grid_ref):
    uw = uw_ref[...]
    uh = uh_ref[...]
    ud = ud_ref[...]

    rj = lax.broadcasted_iota(jnp.int32, (K, K + 1), 0)
    ck = lax.broadcasted_iota(jnp.int32, (K, K + 1), 1)
    tri = (rj < ck).astype(jnp.float32)

    col31 = lax.broadcasted_iota(jnp.int32, (V, K + 1), 1)

    def knots(u):
        m = jnp.max(u, axis=-1, keepdims=True)
        e = jnp.exp(u - m)
        sm = e / jnp.sum(e, axis=-1, keepdims=True)
        frac = MIN_BIN_WIDTH + (1.0 - MIN_BIN_WIDTH * K) * sm
        c = lax.dot_general(frac, tri, (((1,), (0,)), ((), ())),
                            precision=lax.Precision.HIGHEST,
                            preferred_element_type=jnp.float32)
        c = jnp.where(col31 == K, 1.0, c)
        return c, c[:, 1:] - c[:, :-1]

    cw, w = knots(uw)
    ch, h = knots(uh)

    const = math.log(math.exp(1.0 - MIN_DERIVATIVE) - 1.0)
    ud_p = jnp.concatenate(
        [jnp.full((V, 1), const, jnp.float32), ud,
         jnp.full((V, 1), const, jnp.float32)], axis=1)
    deriv = MIN_DERIVATIVE + (jnp.log1p(jnp.exp(-jnp.abs(ud_p)))
                              + jnp.maximum(ud_p, 0.0))

    locs = jnp.where(col31 == K, 1.0 + EPS, cw)

    gp = lax.broadcasted_iota(jnp.int32, (V, GP), 1).astype(jnp.float32)
    gp = gp * (1.0 / GP)
    g = jnp.zeros((V, GP), jnp.int32)
    for k in range(K + 1):
        g = g + (gp >= locs[:, k:k + 1]).astype(jnp.int32)
    g = g - 1
    rows = lax.broadcasted_iota(jnp.int32, (V, GP), 0)
    grid_ref[...] = g * V + rows

    locs_ref[...] = jnp.concatenate(
        [locs, jnp.full((V, 1), 2.0, jnp.float32)], 1)
    cw_ref[...] = jnp.concatenate([cw, jnp.ones((V, 1), jnp.float32)], 1)
    ch_ref[...] = jnp.concatenate([ch, jnp.ones((V, 1), jnp.float32)], 1)
    rw_ref[...] = jnp.concatenate([1.0 / w, jnp.ones((V, 2), jnp.float32)], 1)
    h_ref[...] = jnp.concatenate([h, jnp.ones((V, 2), jnp.float32)], 1)
    d_ref[...] = jnp.concatenate([deriv, jnp.ones((V, 1), jnp.float32)], 1)


_t32 = jax.ShapeDtypeStruct((V, V), jnp.float32)


def _make_tables(uw, uh, ud):
    return pl.pallas_call(
        _tables_body,
        out_shape=(_t32,) * 6 + (jax.ShapeDtypeStruct((V, GP), jnp.int32),),
    )(uw, uh, ud)


def _ln16(r):
    bits = plsc.bitcast(r, jnp.int32)
    e = lax.shift_right_arithmetic(bits, 23) - 127
    m = plsc.bitcast((bits & 0x007FFFFF) | 0x3F800000, jnp.float32)
    t = m - 1.0
    p = jnp.full((16,), LNC[6], jnp.float32)
    for c in LNC[5::-1]:
        p = p * t + c
    return e.astype(jnp.float32) * LN2 + p


def _sc_body(x_hbm, tab_hbm, grid_hbm,
             out_hbm, ld_hbm,
             x_v0, x_v1, o_v0, o_v1, l_v0, l_v1,
             tab_v, grid_v,
             sem_x0, sem_x1, sem_o0, sem_o1, sem_l0, sem_l1):
    wid = lax.axis_index("s") * 2 + lax.axis_index("c")
    base = wid * RT

    x_bufs = (x_v0, x_v1)
    o_bufs = (o_v0, o_v1)
    l_bufs = (l_v0, l_v1)
    sx = (sem_x0, sem_x1)
    so = (sem_o0, sem_o1)
    sl = (sem_l0, sem_l1)

    h_x, h_o, h_l = {}, {}, {}
    h_x[0] = pltpu.async_copy(x_hbm.at[pl.ds(base, RS)], x_bufs[0], sx[0])

    pltpu.sync_copy(tab_hbm, tab_v)
    pltpu.sync_copy(grid_hbm, grid_v)
    T = V * V
    locs_v = tab_v.at[pl.ds(0, T)]
    cw_v = tab_v.at[pl.ds(T, T)]
    rw_v = tab_v.at[pl.ds(2 * T, T)]
    ch_v = tab_v.at[pl.ds(3 * T, T)]
    h_v = tab_v.at[pl.ds(4 * T, T)]
    d_v = tab_v.at[pl.ds(5 * T, T)]

    iota16 = lax.iota(jnp.int32, 16)

    for st in range(N_STAGES):
        bi = st % 2
        off = base + st * RS
        if st + 1 < N_STAGES:
            nb = (st + 1) % 2
            h_x[st + 1] = pltpu.async_copy(
                x_hbm.at[pl.ds(off + RS, RS)], x_bufs[nb], sx[nb])
        h_x[st].wait()
        if st >= 2:
            h_o[st - 2].wait()
            h_l[st - 2].wait()
        x_v = x_bufs[bi]
        o_v = o_bufs[bi]
        l_v = l_bufs[bi]

        @plsc.parallel_loop(0, S, step=16, unroll=4)
        def inner(i):
            r = i >> 5
            c = i & 16
            x = x_v[r, pl.ds(c, 16)]
            v = iota16 + c

            s = (x * float(GP)).astype(jnp.int32)
            lo0 = plsc.load_gather(grid_v, [(s << 5) + v])
            t2 = lo0 + 2 * V
            g2 = plsc.load_gather(locs_v, [t2])
            lo = jnp.where(x >= g2, t2, lo0)
            t1 = lo + V
            g1 = plsc.load_gather(locs_v, [t1])
            lo = jnp.where(x >= g1, t1, lo)

            icw = plsc.load_gather(cw_v, [lo])
            irw = plsc.load_gather(rw_v, [lo])
            ich = plsc.load_gather(ch_v, [lo])
            ih = plsc.load_gather(h_v, [lo])
            id0 = plsc.load_gather(d_v, [lo])
            id1 = plsc.load_gather(d_v, [lo + V])

            idl = ih * irw
            th = (x - icw) * irw
            th2 = th * th
            omt = 1.0 - th
            tomt = th * omt
            num = ih * (idl * th2 + id0 * tomt)
            den = idl + (id0 + id1 - 2.0 * idl) * tomt
            rden = 1.0 / den
            dn = (idl * idl) * (id1 * th2 + 2.0 * idl * tomt
                                + id0 * (omt * omt))
            o_v[r, pl.ds(c, 16)] = ich + num * rden
            l_v[r, pl.ds(c, 16)] = _ln16(dn * rden * rden)

        h_o[st] = pltpu.async_copy(o_v, out_hbm.at[pl.ds(off, RS)], so[bi])
        h_l[st] = pltpu.async_copy(l_v, ld_hbm.at[pl.ds(off, RS)], sl[bi])

    for st in range(max(0, N_STAGES - 2), N_STAGES):
        h_o[st].wait()
        h_l[st].wait()


_sc_call = functools.partial(
    pl.kernel,
    mesh=plsc.VectorSubcoreMesh(core_axis_name="c", subcore_axis_name="s"),
    compiler_params=pltpu.CompilerParams(needs_layout_passes=False),
    out_type=(jax.ShapeDtypeStruct((B, V), jnp.float32),
              jax.ShapeDtypeStruct((B, V), jnp.float32)),
    scratch_types=[
        pltpu.VMEM((RS, V), jnp.float32),
        pltpu.VMEM((RS, V), jnp.float32),
        pltpu.VMEM((RS, V), jnp.float32),
        pltpu.VMEM((RS, V), jnp.float32),
        pltpu.VMEM((RS, V), jnp.float32),
        pltpu.VMEM((RS, V), jnp.float32),
        pltpu.VMEM((6 * V * V,), jnp.float32),
        pltpu.VMEM((V * GP,), jnp.int32),
        pltpu.SemaphoreType.DMA,
        pltpu.SemaphoreType.DMA,
        pltpu.SemaphoreType.DMA,
        pltpu.SemaphoreType.DMA,
        pltpu.SemaphoreType.DMA,
        pltpu.SemaphoreType.DMA,
    ],
)(_sc_body)


def kernel(inputs, unnormalized_widths, unnormalized_heights,
           unnormalized_derivatives):
    *tab6, grid = _make_tables(
        unnormalized_widths, unnormalized_heights, unnormalized_derivatives)
    tab = jnp.stack([t.T for t in tab6]).reshape(-1)
    return _sc_call(inputs, tab, grid.T.reshape(-1))
